# Initial kernel scaffold; baseline (speedup 1.0000x reference)
import jax
import jax.numpy as jnp
from jax.experimental import pallas as pl


def kernel(feats, edge_index, W, b, prelu_a, att_W, att_b, att_q):
    raise NotImplementedError("write your pallas kernel here")


# speedup vs baseline: 5.6829x; 5.6829x over previous
import functools

import jax
import jax.numpy as jnp
from jax.experimental import pallas as pl
from jax.experimental.pallas import tpu as pltpu
from jax.experimental.pallas import tpu_sc as plsc

_NC = 2
_NS = 16
_CB = 80
_INNER = 10


def _sc_aggregate(feats_flat, idx_packed, zacc, M, N, E):
    MN, D = feats_flat.shape
    BLK = _CB * _INNER
    NBLK = E // BLK
    BPT = NBLK // _NS
    MPC = M // _NC
    RPT = (N // _NS) & ~7
    RLAST = N - RPT * (_NS - 1)

    mesh = plsc.VectorSubcoreMesh(core_axis_name="c", subcore_axis_name="s",
                                  num_cores=_NC, num_subcores=_NS)

    def body(feats_hbm, idx_hbm, zacc_hbm,
             acc_out, deg_out, buf, src2d, dst2d, rows, ones1, deg_t,
             acc_s, deg_s, gsem):
        c = jax.lax.axis_index("c")
        s = jax.lax.axis_index("s")
        r0 = s * RPT
        last = s == (_NS - 1)
        onev = jnp.full((16,), 1.0, jnp.float32)
        zerov = jnp.zeros((16,), jnp.float32)
        for r in range(_CB // 16):
            ones1[pl.ds(r * 16, 16)] = onev

        def zero_slices():
            @pl.when(jnp.logical_not(last))
            def _():
                pltpu.sync_copy(zacc_hbm.at[pl.ds(0, RPT)],
                                acc_s.at[pl.ds(r0, RPT)])

            @pl.when(last)
            def _():
                pltpu.sync_copy(zacc_hbm.at[pl.ds(0, RLAST)],
                                acc_s.at[pl.ds(r0, RLAST)])

            for k in range(RLAST // 16):
                deg_t[pl.ds(k * 16, 16)] = zerov

            @pl.when(jnp.logical_not(last))
            def _():
                pltpu.sync_copy(deg_t.at[pl.ds(0, RPT)],
                                deg_s.at[pl.ds(r0, RPT)])

            @pl.when(last)
            def _():
                pltpu.sync_copy(deg_t.at[pl.ds(0, RLAST)],
                                deg_s.at[pl.ds(r0, RLAST)])

        zero_slices()
        for j in range(MPC):
            m = c * MPC + j
            m_off = m * N
            plsc.subcore_barrier()

            def outer(kb, carry):
                blk = m * NBLK + s * BPT + kb
                pltpu.sync_copy(idx_hbm.at[pl.ds(blk * 2 * BLK, 2 * BLK)],
                                buf)
                for t in range(_INNER):
                    for k in range(_CB // 16):
                        sl16 = pl.ds(k * 16, 16)
                        src2d[t, sl16] = (
                            buf[pl.ds(t * _CB + k * 16, 16)] + m_off)
                        dst2d[t, sl16] = (
                            buf[pl.ds(BLK + t * _CB + k * 16, 16)])
                for t in range(_INNER):
                    pltpu.async_copy(feats_hbm.at[src2d.at[t]], rows,
                                     gsem).wait()
                    pltpu.sync_copy(rows, acc_s.at[dst2d.at[t]], add=True)
                    pltpu.sync_copy(ones1, deg_s.at[dst2d.at[t]], add=True)
                return carry

            jax.lax.fori_loop(0, BPT, outer, 0)
            plsc.subcore_barrier()

            @pl.when(jnp.logical_not(last))
            def _():
                pltpu.sync_copy(acc_s.at[pl.ds(r0, RPT)],
                                acc_out.at[m, pl.ds(r0, RPT)])
                pltpu.sync_copy(deg_s.at[pl.ds(r0, RPT)],
                                deg_t.at[pl.ds(0, RPT)])
                pltpu.sync_copy(deg_t.at[pl.ds(0, RPT)],
                                deg_out.at[pl.ds(m * N + r0, RPT)])

            @pl.when(last)
            def _():
                pltpu.sync_copy(acc_s.at[pl.ds(r0, RLAST)],
                                acc_out.at[m, pl.ds(r0, RLAST)])
                pltpu.sync_copy(deg_s.at[pl.ds(r0, RLAST)],
                                deg_t.at[pl.ds(0, RLAST)])
                pltpu.sync_copy(deg_t.at[pl.ds(0, RLAST)],
                                deg_out.at[pl.ds(m * N + r0, RLAST)])

            if j + 1 < MPC:
                zero_slices()

    fn = pl.kernel(
        body,
        out_type=(
            jax.ShapeDtypeStruct((M, N, D), jnp.float32),
            jax.ShapeDtypeStruct((M * N,), jnp.float32),
        ),
        mesh=mesh,
        scratch_types=[
            pltpu.VMEM((2 * _CB * _INNER,), jnp.int32),
            pltpu.VMEM((_INNER, _CB), jnp.int32),
            pltpu.VMEM((_INNER, _CB), jnp.int32),
            pltpu.VMEM((_CB, D), jnp.float32),
            pltpu.VMEM((_CB,), jnp.float32),
            pltpu.VMEM((RLAST,), jnp.float32),
            pltpu.VMEM_SHARED((N, D), jnp.float32),
            pltpu.VMEM_SHARED((N,), jnp.float32),
            pltpu.SemaphoreType.DMA,
        ],
    )
    return fn(feats_flat, idx_packed, zacc)


def _dense_stage1(acc, deg, W, b3, pa, attW, attb2, R):
    M, N, D = acc.shape
    H = W.shape[2]

    def body(acc_ref, deg_ref, w_ref, b_ref, pa_ref, aw_ref, ab_ref,
             h_ref, ws_ref):
        i = pl.program_id(0)

        @pl.when(i == 0)
        def _init():
            ws_ref[...] = jnp.zeros_like(ws_ref)

        for m in range(M):
            a = acc_ref[m]
            dg = deg_ref[m]
            x = a / jnp.maximum(dg, 1.0)
            y = jnp.dot(x, w_ref[m], preferred_element_type=jnp.float32)
            y = y + b_ref[m]
            hm = jnp.where(y >= 0.0, y, pa_ref[m] * y)
            h_ref[m] = hm
            sm = jnp.tanh(
                jnp.dot(hm, aw_ref[...], preferred_element_type=jnp.float32)
                + ab_ref[...])
            ws_ref[m] += jnp.sum(sm, axis=0, keepdims=True)

    return pl.pallas_call(
        body,
        grid=(N // R,),
        in_specs=[
            pl.BlockSpec((M, R, D), lambda i: (0, i, 0)),
            pl.BlockSpec((M, R, 1), lambda i: (0, i, 0)),
            pl.BlockSpec((M, D, H), lambda i: (0, 0, 0)),
            pl.BlockSpec((M, 1, H), lambda i: (0, 0, 0)),
            pl.BlockSpec(memory_space=pltpu.SMEM),
            pl.BlockSpec((D, H), lambda i: (0, 0)),
            pl.BlockSpec((1, H), lambda i: (0, 0)),
        ],
        out_specs=[
            pl.BlockSpec((M, R, H), lambda i: (0, i, 0)),
            pl.BlockSpec((M, 1, H), lambda i: (0, 0, 0)),
        ],
        out_shape=[
            jax.ShapeDtypeStruct((M, N, H), jnp.float32),
            jax.ShapeDtypeStruct((M, 1, H), jnp.float32),
        ],
    )(acc, deg, W, b3, pa, attW, attb2)


def _dense_stage2(h, ws, attq2, R):
    M, N, H = h.shape
    inv_n = 1.0 / N

    def body(h_ref, ws_ref, q_ref, z_ref):
        wsum = ws_ref[...][:, 0, :]
        wv = jnp.sum(wsum * q_ref[...], axis=1, keepdims=True) * inv_n
        e = jnp.exp(wv - jnp.max(wv, axis=0, keepdims=True))
        beta = e / jnp.sum(e, axis=0, keepdims=True)
        z_ref[...] = jnp.sum(h_ref[...] * beta[:, :, None], axis=0)

    return pl.pallas_call(
        body,
        grid=(N // R,),
        in_specs=[
            pl.BlockSpec((M, R, H), lambda i: (0, i, 0)),
            pl.BlockSpec((M, 1, H), lambda i: (0, 0, 0)),
            pl.BlockSpec((1, H), lambda i: (0, 0)),
        ],
        out_specs=pl.BlockSpec((R, H), lambda i: (i, 0)),
        out_shape=jax.ShapeDtypeStruct((N, H), jnp.float32),
    )(h, ws, attq2)


@jax.jit
def kernel(feats, edge_index, W, b, prelu_a, att_W, att_b, att_q):
    M, N, D = feats.shape
    H = W.shape[2]
    E = edge_index.shape[2]

    feats_flat = feats.reshape(M * N, D)
    BLK = _CB * _INNER
    idx_packed = edge_index.reshape(M, 2, E // BLK, BLK)
    idx_packed = idx_packed.transpose(0, 2, 1, 3).reshape(M * E * 2)
    rmax = N - ((N // _NS) & ~7) * (_NS - 1)
    zacc = jnp.zeros((rmax, D), jnp.float32)

    acc, deg_flat = _sc_aggregate(feats_flat, idx_packed, zacc, M, N, E)

    h, ws = _dense_stage1(acc, deg_flat.reshape(M, N, 1), W,
                          b.reshape(M, 1, H), prelu_a,
                          att_W, att_b.reshape(1, H), R=1000)
    z = _dense_stage2(h, ws, att_q.reshape(1, H), R=1000)
    return z


# speedup vs baseline: 8.4543x; 1.4877x over previous
import functools

import jax
import jax.numpy as jnp
from jax.experimental import pallas as pl
from jax.experimental.pallas import tpu as pltpu
from jax.experimental.pallas import tpu_sc as plsc

_NC = 2
_NS = 16
_CB = 80
_INNER = 10


def _sc_aggregate(feats_flat, idx_packed, zacc, M, N, E):
    MN, D = feats_flat.shape
    BLK = _CB * _INNER
    NBLK = E // BLK
    BPT = NBLK // _NS
    MPC = M // _NC
    RPT = (N // _NS) & ~7
    RLAST = N - RPT * (_NS - 1)

    mesh = plsc.VectorSubcoreMesh(core_axis_name="c", subcore_axis_name="s",
                                  num_cores=_NC, num_subcores=_NS)

    def body(feats_hbm, idx_hbm, zacc_hbm,
             acc_out, deg_out, buf, src2d, dst2d, rows2, ones1, deg_t,
             acc_s, deg_s, gsem0, gsem1):
        c = jax.lax.axis_index("c")
        s = jax.lax.axis_index("s")
        r0 = s * RPT
        last = s == (_NS - 1)
        onev = jnp.full((16,), 1.0, jnp.float32)
        zerov = jnp.zeros((16,), jnp.float32)
        for r in range(_CB // 16):
            ones1[pl.ds(r * 16, 16)] = onev

        def zero_slices():
            @pl.when(jnp.logical_not(last))
            def _():
                pltpu.sync_copy(zacc_hbm.at[pl.ds(0, RPT)],
                                acc_s.at[pl.ds(r0, RPT)])

            @pl.when(last)
            def _():
                pltpu.sync_copy(zacc_hbm.at[pl.ds(0, RLAST)],
                                acc_s.at[pl.ds(r0, RLAST)])

            for k in range(RLAST // 16):
                deg_t[pl.ds(k * 16, 16)] = zerov

            @pl.when(jnp.logical_not(last))
            def _():
                pltpu.sync_copy(deg_t.at[pl.ds(0, RPT)],
                                deg_s.at[pl.ds(r0, RPT)])

            @pl.when(last)
            def _():
                pltpu.sync_copy(deg_t.at[pl.ds(0, RLAST)],
                                deg_s.at[pl.ds(r0, RLAST)])

        zero_slices()
        for j in range(MPC):
            m = c * MPC + j
            m_off = m * N
            plsc.subcore_barrier()

            def outer(kb, carry):
                blk = m * NBLK + s * BPT + kb
                pltpu.sync_copy(idx_hbm.at[pl.ds(blk * 2 * BLK, 2 * BLK)],
                                buf)
                for t in range(_INNER):
                    for k in range(_CB // 16):
                        sl16 = pl.ds(k * 16, 16)
                        src2d[t, sl16] = (
                            buf[pl.ds(t * _CB + k * 16, 16)] + m_off)
                        dst2d[t, sl16] = (
                            buf[pl.ds(BLK + t * _CB + k * 16, 16)])
                sems = (gsem0, gsem1)
                cps = [pltpu.async_copy(feats_hbm.at[src2d.at[0]],
                                        rows2.at[0], gsem0), None]
                for t in range(_INNER):
                    if t + 1 < _INNER:
                        cps[(t + 1) % 2] = pltpu.async_copy(
                            feats_hbm.at[src2d.at[t + 1]],
                            rows2.at[(t + 1) % 2], sems[(t + 1) % 2])
                    cps[t % 2].wait()
                    pltpu.sync_copy(rows2.at[t % 2],
                                    acc_s.at[dst2d.at[t]], add=True)
                    pltpu.sync_copy(ones1, deg_s.at[dst2d.at[t]], add=True)
                return carry

            jax.lax.fori_loop(0, BPT, outer, 0)
            plsc.subcore_barrier()

            @pl.when(jnp.logical_not(last))
            def _():
                pltpu.sync_copy(acc_s.at[pl.ds(r0, RPT)],
                                acc_out.at[m, pl.ds(r0, RPT)])
                pltpu.sync_copy(deg_s.at[pl.ds(r0, RPT)],
                                deg_t.at[pl.ds(0, RPT)])
                pltpu.sync_copy(deg_t.at[pl.ds(0, RPT)],
                                deg_out.at[pl.ds(m * N + r0, RPT)])

            @pl.when(last)
            def _():
                pltpu.sync_copy(acc_s.at[pl.ds(r0, RLAST)],
                                acc_out.at[m, pl.ds(r0, RLAST)])
                pltpu.sync_copy(deg_s.at[pl.ds(r0, RLAST)],
                                deg_t.at[pl.ds(0, RLAST)])
                pltpu.sync_copy(deg_t.at[pl.ds(0, RLAST)],
                                deg_out.at[pl.ds(m * N + r0, RLAST)])

            if j + 1 < MPC:
                zero_slices()

    fn = pl.kernel(
        body,
        out_type=(
            jax.ShapeDtypeStruct((M, N, D), jnp.float32),
            jax.ShapeDtypeStruct((M * N,), jnp.float32),
        ),
        mesh=mesh,
        scratch_types=[
            pltpu.VMEM((2 * _CB * _INNER,), jnp.int32),
            pltpu.VMEM((_INNER, _CB), jnp.int32),
            pltpu.VMEM((_INNER, _CB), jnp.int32),
            pltpu.VMEM((2, _CB, D), jnp.float32),
            pltpu.VMEM((_CB,), jnp.float32),
            pltpu.VMEM((RLAST,), jnp.float32),
            pltpu.VMEM_SHARED((N, D), jnp.float32),
            pltpu.VMEM_SHARED((N,), jnp.float32),
            pltpu.SemaphoreType.DMA,
            pltpu.SemaphoreType.DMA,
        ],
    )
    return fn(feats_flat, idx_packed, zacc)


def _dense_stage1(acc, deg, W, b3, pa, attW, attb2, R):
    M, N, D = acc.shape
    H = W.shape[2]

    def body(acc_ref, deg_ref, w_ref, b_ref, pa_ref, aw_ref, ab_ref,
             h_ref, ws_ref):
        i = pl.program_id(0)

        @pl.when(i == 0)
        def _init():
            ws_ref[...] = jnp.zeros_like(ws_ref)

        for m in range(M):
            a = acc_ref[m]
            dg = deg_ref[m]
            x = a / jnp.maximum(dg, 1.0)
            y = jnp.dot(x, w_ref[m], preferred_element_type=jnp.float32)
            y = y + b_ref[m]
            hm = jnp.where(y >= 0.0, y, pa_ref[m] * y)
            h_ref[m] = hm
            sm = jnp.tanh(
                jnp.dot(hm, aw_ref[...], preferred_element_type=jnp.float32)
                + ab_ref[...])
            ws_ref[m] += jnp.sum(sm, axis=0, keepdims=True)

    return pl.pallas_call(
        body,
        grid=(N // R,),
        in_specs=[
            pl.BlockSpec((M, R, D), lambda i: (0, i, 0)),
            pl.BlockSpec((M, R, 1), lambda i: (0, i, 0)),
            pl.BlockSpec((M, D, H), lambda i: (0, 0, 0)),
            pl.BlockSpec((M, 1, H), lambda i: (0, 0, 0)),
            pl.BlockSpec(memory_space=pltpu.SMEM),
            pl.BlockSpec((D, H), lambda i: (0, 0)),
            pl.BlockSpec((1, H), lambda i: (0, 0)),
        ],
        out_specs=[
            pl.BlockSpec((M, R, H), lambda i: (0, i, 0)),
            pl.BlockSpec((M, 1, H), lambda i: (0, 0, 0)),
        ],
        out_shape=[
            jax.ShapeDtypeStruct((M, N, H), jnp.float32),
            jax.ShapeDtypeStruct((M, 1, H), jnp.float32),
        ],
    )(acc, deg, W, b3, pa, attW, attb2)


def _dense_stage2(h, ws, attq2, R):
    M, N, H = h.shape
    inv_n = 1.0 / N

    def body(h_ref, ws_ref, q_ref, z_ref):
        wsum = ws_ref[...][:, 0, :]
        wv = jnp.sum(wsum * q_ref[...], axis=1, keepdims=True) * inv_n
        e = jnp.exp(wv - jnp.max(wv, axis=0, keepdims=True))
        beta = e / jnp.sum(e, axis=0, keepdims=True)
        z_ref[...] = jnp.sum(h_ref[...] * beta[:, :, None], axis=0)

    return pl.pallas_call(
        body,
        grid=(N // R,),
        in_specs=[
            pl.BlockSpec((M, R, H), lambda i: (0, i, 0)),
            pl.BlockSpec((M, 1, H), lambda i: (0, 0, 0)),
            pl.BlockSpec((1, H), lambda i: (0, 0)),
        ],
        out_specs=pl.BlockSpec((R, H), lambda i: (i, 0)),
        out_shape=jax.ShapeDtypeStruct((N, H), jnp.float32),
    )(h, ws, attq2)


@jax.jit
def kernel(feats, edge_index, W, b, prelu_a, att_W, att_b, att_q):
    M, N, D = feats.shape
    H = W.shape[2]
    E = edge_index.shape[2]

    feats_flat = feats.reshape(M * N, D)
    BLK = _CB * _INNER
    idx_packed = edge_index.reshape(M, 2, E // BLK, BLK)
    idx_packed = idx_packed.transpose(0, 2, 1, 3).reshape(M * E * 2)
    rmax = N - ((N // _NS) & ~7) * (_NS - 1)
    zacc = jnp.zeros((rmax, D), jnp.float32)

    acc, deg_flat = _sc_aggregate(feats_flat, idx_packed, zacc, M, N, E)

    h, ws = _dense_stage1(acc, deg_flat.reshape(M, N, 1), W,
                          b.reshape(M, 1, H), prelu_a,
                          att_W, att_b.reshape(1, H), R=1000)
    z = _dense_stage2(h, ws, att_q.reshape(1, H), R=1000)
    return z


# speedup vs baseline: 8.7451x; 1.0344x over previous
import functools

import jax
import jax.numpy as jnp
from jax.experimental import pallas as pl
from jax.experimental.pallas import tpu as pltpu
from jax.experimental.pallas import tpu_sc as plsc

_NC = 2
_NS = 16
_CB = 80
_INNER = 25


def _sc_aggregate(feats_flat, idx_packed, zacc, ones_e, zeros_e, M, N, E):
    MN, D = feats_flat.shape
    BLK = _CB * _INNER
    NBLK = E // BLK
    BPT = NBLK // _NS
    MPC = M // _NC
    RPT = (N // _NS) & ~7
    RLAST = N - RPT * (_NS - 1)

    mesh = plsc.VectorSubcoreMesh(core_axis_name="c", subcore_axis_name="s",
                                  num_cores=_NC, num_subcores=_NS)

    def body(feats_hbm, idx_hbm, zacc_hbm, ones_hbm, zeros_hbm,
             acc_out, deg_out, buf, src2d, dst2d, rows_a, rows_b, ones1d,
             stage1d, acc_s, deg_s, gsem0, gsem1):
        c = jax.lax.axis_index("c")
        s = jax.lax.axis_index("s")
        r0 = s * RPT
        last = s == (_NS - 1)

        pltpu.sync_copy(ones_hbm, ones1d)

        def zero_acc():
            @pl.when(jnp.logical_not(last))
            def _():
                pltpu.sync_copy(zacc_hbm.at[pl.ds(0, RPT)],
                                acc_s.at[pl.ds(r0, RPT)])

            @pl.when(last)
            def _():
                pltpu.sync_copy(zacc_hbm.at[pl.ds(0, RLAST)],
                                acc_s.at[pl.ds(r0, RLAST)])

        def zero_deg():
            pltpu.sync_copy(zeros_hbm, stage1d)

            @pl.when(jnp.logical_not(last))
            def _():
                pltpu.sync_copy(stage1d.at[pl.ds(0, RPT)],
                                deg_s.at[pl.ds(r0, RPT)])

            @pl.when(last)
            def _():
                pltpu.sync_copy(stage1d, deg_s.at[pl.ds(r0, RLAST)])

        zero_acc()
        zero_deg()
        for j in range(MPC):
            m = c * MPC + j
            m_off = m * N
            plsc.subcore_barrier()

            def outer(kb, carry):
                blk = m * NBLK + s * BPT + kb
                pltpu.sync_copy(idx_hbm.at[pl.ds(blk * 2 * BLK, 2 * BLK)],
                                buf)
                for t in range(_INNER):
                    for k in range(_CB // 16):
                        sl16 = pl.ds(k * 16, 16)
                        src2d[t, sl16] = (
                            buf[pl.ds(t * _CB + k * 16, 16)] + m_off)
                        dst2d[t, sl16] = buf[pl.ds(BLK + t * _CB + k * 16,
                                                   16)]
                    pltpu.sync_copy(ones1d, deg_s.at[dst2d.at[t]], add=True)
                sems = (gsem0, gsem1)
                rbufs = (rows_a, rows_b)
                cps = [pltpu.async_copy(feats_hbm.at[src2d.at[0]],
                                        rows_a, gsem0), None]
                for t in range(_INNER):
                    if t + 1 < _INNER:
                        cps[(t + 1) % 2] = pltpu.async_copy(
                            feats_hbm.at[src2d.at[t + 1]],
                            rbufs[(t + 1) % 2], sems[(t + 1) % 2])
                    cps[t % 2].wait()
                    pltpu.sync_copy(rbufs[t % 2],
                                    acc_s.at[dst2d.at[t]], add=True)
                return carry

            jax.lax.fori_loop(0, BPT, outer, 0)
            plsc.subcore_barrier()

            @pl.when(jnp.logical_not(last))
            def _():
                pltpu.sync_copy(acc_s.at[pl.ds(r0, RPT)],
                                acc_out.at[m, pl.ds(r0, RPT)])
                pltpu.sync_copy(deg_s.at[pl.ds(r0, RPT)],
                                stage1d.at[pl.ds(0, RPT)])
                pltpu.sync_copy(stage1d.at[pl.ds(0, RPT)],
                                deg_out.at[pl.ds(m_off + r0, RPT)])

            @pl.when(last)
            def _():
                pltpu.sync_copy(acc_s.at[pl.ds(r0, RLAST)],
                                acc_out.at[m, pl.ds(r0, RLAST)])
                pltpu.sync_copy(deg_s.at[pl.ds(r0, RLAST)], stage1d)
                pltpu.sync_copy(stage1d, deg_out.at[pl.ds(m_off + r0, RLAST)])

            if j + 1 < MPC:
                zero_acc()
                zero_deg()

    fn = pl.kernel(
        body,
        out_type=(
            jax.ShapeDtypeStruct((M, N, D), jnp.float32),
            jax.ShapeDtypeStruct((M * N,), jnp.float32),
        ),
        mesh=mesh,
        scratch_types=[
            pltpu.VMEM((2 * _CB * _INNER,), jnp.int32),
            pltpu.VMEM((_INNER, _CB), jnp.int32),
            pltpu.VMEM((_INNER, _CB), jnp.int32),
            pltpu.VMEM((_CB, D), jnp.float32),
            pltpu.VMEM((_CB, D), jnp.float32),
            pltpu.VMEM((_CB,), jnp.float32),
            pltpu.VMEM((N - ((N // _NS) & ~7) * (_NS - 1),), jnp.float32),
            pltpu.VMEM_SHARED((N, D), jnp.float32),
            pltpu.VMEM_SHARED((N,), jnp.float32),
            pltpu.SemaphoreType.DMA,
            pltpu.SemaphoreType.DMA,
        ],
    )
    return fn(feats_flat, idx_packed, zacc, ones_e, zeros_e)


def _dense_stage1(acc, deg, W, b3, pa, attW, attb2, R):
    M, N, D = acc.shape
    H = W.shape[2]

    def body(acc_ref, deg_ref, w_ref, b_ref, pa_ref, aw_ref, ab_ref,
             h_ref, ws_ref):
        i = pl.program_id(0)

        @pl.when(i == 0)
        def _init():
            ws_ref[...] = jnp.zeros_like(ws_ref)

        for m in range(M):
            a = acc_ref[m]
            dg = deg_ref[m]
            x = a / jnp.maximum(dg, 1.0)
            y = jnp.dot(x, w_ref[m], preferred_element_type=jnp.float32)
            y = y + b_ref[m]
            hm = jnp.where(y >= 0.0, y, pa_ref[m] * y)
            h_ref[m] = hm
            sm = jnp.tanh(
                jnp.dot(hm, aw_ref[...], preferred_element_type=jnp.float32)
                + ab_ref[...])
            ws_ref[m] += jnp.sum(sm, axis=0, keepdims=True)

    return pl.pallas_call(
        body,
        grid=(N // R,),
        in_specs=[
            pl.BlockSpec((M, R, D), lambda i: (0, i, 0)),
            pl.BlockSpec((M, R, 1), lambda i: (0, i, 0)),
            pl.BlockSpec((M, D, H), lambda i: (0, 0, 0)),
            pl.BlockSpec((M, 1, H), lambda i: (0, 0, 0)),
            pl.BlockSpec(memory_space=pltpu.SMEM),
            pl.BlockSpec((D, H), lambda i: (0, 0)),
            pl.BlockSpec((1, H), lambda i: (0, 0)),
        ],
        out_specs=[
            pl.BlockSpec((M, R, H), lambda i: (0, i, 0)),
            pl.BlockSpec((M, 1, H), lambda i: (0, 0, 0)),
        ],
        out_shape=[
            jax.ShapeDtypeStruct((M, N, H), jnp.float32),
            jax.ShapeDtypeStruct((M, 1, H), jnp.float32),
        ],
    )(acc, deg, W, b3, pa, attW, attb2)


def _dense_stage2(h, ws, attq2, R):
    M, N, H = h.shape
    inv_n = 1.0 / N

    def body(h_ref, ws_ref, q_ref, z_ref):
        wsum = ws_ref[...][:, 0, :]
        wv = jnp.sum(wsum * q_ref[...], axis=1, keepdims=True) * inv_n
        e = jnp.exp(wv - jnp.max(wv, axis=0, keepdims=True))
        beta = e / jnp.sum(e, axis=0, keepdims=True)
        z_ref[...] = jnp.sum(h_ref[...] * beta[:, :, None], axis=0)

    return pl.pallas_call(
        body,
        grid=(N // R,),
        in_specs=[
            pl.BlockSpec((M, R, H), lambda i: (0, i, 0)),
            pl.BlockSpec((M, 1, H), lambda i: (0, 0, 0)),
            pl.BlockSpec((1, H), lambda i: (0, 0)),
        ],
        out_specs=pl.BlockSpec((R, H), lambda i: (i, 0)),
        out_shape=jax.ShapeDtypeStruct((N, H), jnp.float32),
    )(h, ws, attq2)


@jax.jit
def kernel(feats, edge_index, W, b, prelu_a, att_W, att_b, att_q):
    M, N, D = feats.shape
    H = W.shape[2]
    E = edge_index.shape[2]

    feats_flat = feats.reshape(M * N, D)
    BLK = _CB * _INNER
    idx_packed = edge_index.reshape(M, 2, E // BLK, BLK)
    idx_packed = idx_packed.transpose(0, 2, 1, 3).reshape(M * E * 2)
    rmax = N - ((N // _NS) & ~7) * (_NS - 1)
    zacc = jnp.zeros((rmax, D), jnp.float32)
    ones_e = jnp.ones((_CB,), jnp.float32)
    zeros_e = jnp.zeros((rmax,), jnp.float32)

    acc, degf = _sc_aggregate(feats_flat, idx_packed, zacc, ones_e, zeros_e,
                              M, N, E)
    deg = degf.reshape(M, N)

    h, ws = _dense_stage1(acc, deg.reshape(M, N, 1), W,
                          b.reshape(M, 1, H), prelu_a,
                          att_W, att_b.reshape(1, H), R=1000)
    z = _dense_stage2(h, ws, att_q.reshape(1, H), R=1000)
    return z
